# Initial kernel scaffold; baseline (speedup 1.0000x reference)
import jax
import jax.numpy as jnp
from jax.experimental import pallas as pl


def kernel(predictions, labels):
    raise NotImplementedError("write your pallas kernel here")


# speedup vs baseline: 126.2859x; 126.2859x over previous
import jax
import jax.numpy as jnp
from jax.experimental import pallas as pl
from jax.experimental.pallas import tpu as pltpu

_B, _C, _H, _W = 2, 2, 224, 224
_INF = 1e12
_FALLBACK = (_H + _W) / 4.0


def _neighbor_sum(img):
    zr = jnp.zeros((_B, 1, _W), jnp.float32)
    zc = jnp.zeros((_B, _H, 1), jnp.float32)
    up = jnp.concatenate([img[:, 1:, :], zr], axis=1)
    down = jnp.concatenate([zr, img[:, :-1, :]], axis=1)
    left = jnp.concatenate([img[:, :, 1:], zc], axis=2)
    right = jnp.concatenate([zc, img[:, :, :-1]], axis=2)
    return img + up + down + left + right


def _hd_kernel(p0_ref, p1_ref, lb_ref, out_ref, scratch_ref):
    p0 = p0_ref[...]
    p1 = p1_ref[...]
    img_b = lb_ref[...]
    img_a = (p1 > p0).astype(jnp.float32)

    boundary_a = img_a * (5.0 - _neighbor_sum(img_a)) * (1.0 - img_b)
    boundary_b = img_b * (5.0 - _neighbor_sum(img_b))
    ma = (boundary_a > 0.0).astype(jnp.float32)
    mb = (boundary_b > 0.0).astype(jnp.float32)

    g = jnp.where(mb > 0.0, jnp.float32(0.0), _INF)

    iot = jax.lax.broadcasted_iota(jnp.int32, (1, _H, 1), 1).astype(jnp.float32)

    scratch_ref[...] = g

    def body(k, run):
        kf = jnp.float32(1.0) * k.astype(jnp.float32)
        dk = iot - kf
        kcol = dk * dk
        row = scratch_ref[:, pl.ds(k, 1), :]
        return jnp.minimum(run, kcol + row)

    init = jnp.full((_B, _H, _W), jnp.float32(2e12), jnp.float32)
    d1 = jax.lax.fori_loop(0, _H, body, init)

    d1t = jnp.transpose(d1, (0, 2, 1))
    scratch_ref[...] = d1t
    d2t = jax.lax.fori_loop(0, _W, body, init)

    mat = jnp.transpose(ma, (0, 2, 1))
    m = jnp.where(mat > 0.0, d2t, -_INF)
    hd2 = jnp.max(jnp.max(m, axis=2), axis=1, keepdims=True)
    any_a = jnp.max(jnp.max(ma, axis=2), axis=1, keepdims=True)
    any_b = jnp.max(jnp.max(mb, axis=2), axis=1, keepdims=True)

    hd = jnp.sqrt(jnp.maximum(hd2, 0.0))
    has_both = (any_a > 0.0) & (any_b > 0.0)
    val = jnp.where(has_both, hd, _FALLBACK)

    lane = jax.lax.broadcasted_iota(jnp.int32, (_B, 128), 1)
    out_ref[...] = jnp.where(lane == 0, val, jnp.float32(0.0))


def kernel(predictions, labels):
    p0 = predictions[:, 0]
    p1 = predictions[:, 1]
    lbf = labels[:, 1].astype(jnp.float32)

    res = pl.pallas_call(
        _hd_kernel,
        out_shape=jax.ShapeDtypeStruct((_B, 128), jnp.float32),
        scratch_shapes=[pltpu.VMEM((_B, _H, _W), jnp.float32)],
    )(p0, p1, lbf)

    val = res[:, 0]
    HD = jnp.zeros((_B, _C + 2), jnp.float32)
    HD = HD.at[:, 1].set(val)
    HD = HD.at[:, 2].set(val * 0.5)
    return HD


# speedup vs baseline: 575.9489x; 4.5607x over previous
import jax
import jax.numpy as jnp
from jax.experimental import pallas as pl
from jax.experimental.pallas import tpu as pltpu

_B, _C, _H, _W = 2, 2, 224, 224
_INF = 1e12
_FALLBACK = (_H + _W) / 4.0


def _neighbor_sum(img):
    zr = jnp.zeros((_B, 1, _W), jnp.float32)
    zc = jnp.zeros((_B, _H, 1), jnp.float32)
    up = jnp.concatenate([img[:, 1:, :], zr], axis=1)
    down = jnp.concatenate([zr, img[:, :-1, :]], axis=1)
    left = jnp.concatenate([img[:, :, 1:], zc], axis=2)
    right = jnp.concatenate([zc, img[:, :, :-1]], axis=2)
    return img + up + down + left + right


def _hd_kernel(p0_ref, p1_ref, lb_ref, out_ref, pad_ref, run_ref):
    p0 = p0_ref[...]
    p1 = p1_ref[...]
    img_b = lb_ref[...]
    img_a = (p1 > p0).astype(jnp.float32)

    boundary_a = img_a * (5.0 - _neighbor_sum(img_a)) * (1.0 - img_b)
    boundary_b = img_b * (5.0 - _neighbor_sum(img_b))
    ma = (boundary_a > 0.0).astype(jnp.float32)
    mb = (boundary_b > 0.0).astype(jnp.float32)

    inf_blk = jnp.full((_B, _H, _W), jnp.float32(_INF), jnp.float32)
    pad_ref[:, 0:_H, :] = inf_blk
    pad_ref[:, 2 * _H:3 * _H, :] = inf_blk

    d = jnp.where(mb > 0.0, jnp.float32(0.0), jnp.float32(_INF))
    for k in (1, 2, 4, 8, 16, 32, 64, 128):
        pad_ref[:, _H:2 * _H, :] = d
        up = pad_ref[:, _H - k:2 * _H - k, :]
        dn = pad_ref[:, _H + k:2 * _H + k, :]
        d = jnp.minimum(d, jnp.minimum(up, dn) + jnp.float32(k))
    d1 = d * d

    d1t = jnp.transpose(d1, (0, 2, 1))
    mat = jnp.transpose(ma, (0, 2, 1))
    pad_ref[:, _H:2 * _H, :] = d1t
    run_ref[...] = d1t

    mm0 = jnp.max(jnp.where(mat > 0.0, d1t, -jnp.float32(_INF)))

    def cond(carry):
        t, mm = carry
        s0 = 8 * t + 1
        return (s0 < _W) & (mm > (s0 * s0).astype(jnp.float32))

    def body(carry):
        t, _ = carry
        wp = pad_ref[:, pl.ds(8 * (_H // 8 - 1 - t), _H + 8), :]
        wm = pad_ref[:, pl.ds(_H + 8 * t, _H + 8), :]
        r = run_ref[...]
        for k in range(8):
            sf = (8 * t + (1 + k)).astype(jnp.float32)
            cand = jnp.minimum(wp[:, 7 - k:7 - k + _H, :],
                               wm[:, 1 + k:1 + k + _H, :])
            r = jnp.minimum(r, cand + sf * sf)
        run_ref[...] = r
        mm = jnp.max(jnp.where(mat > 0.0, r, -jnp.float32(_INF)))
        return t + 1, mm

    jax.lax.while_loop(cond, body, (jnp.int32(0), mm0))

    m = jnp.where(mat > 0.0, run_ref[...], -jnp.float32(_INF))
    hd2 = jnp.max(jnp.max(m, axis=2), axis=1, keepdims=True)
    any_a = jnp.max(jnp.max(ma, axis=2), axis=1, keepdims=True)
    any_b = jnp.max(jnp.max(mb, axis=2), axis=1, keepdims=True)

    hd = jnp.sqrt(jnp.maximum(hd2, 0.0))
    has_both = (any_a > 0.0) & (any_b > 0.0)
    val = jnp.where(has_both, hd, jnp.float32(_FALLBACK))

    lane = jax.lax.broadcasted_iota(jnp.int32, (_B, 128), 1)
    out_ref[...] = jnp.where(lane == 0, val, jnp.float32(0.0))


def kernel(predictions, labels):
    p0 = predictions[:, 0]
    p1 = predictions[:, 1]
    lbf = labels[:, 1].astype(jnp.float32)

    res = pl.pallas_call(
        _hd_kernel,
        out_shape=jax.ShapeDtypeStruct((_B, 128), jnp.float32),
        scratch_shapes=[
            pltpu.VMEM((_B, 3 * _H, _W), jnp.float32),
            pltpu.VMEM((_B, _H, _W), jnp.float32),
        ],
    )(p0, p1, lbf)

    val = res[:, 0]
    HD = jnp.zeros((_B, _C + 2), jnp.float32)
    HD = HD.at[:, 1].set(val)
    HD = HD.at[:, 2].set(val * 0.5)
    return HD


# speedup vs baseline: 1399.2902x; 2.4295x over previous
import jax
import jax.numpy as jnp
from jax.experimental import pallas as pl
from jax.experimental.pallas import tpu as pltpu

_B, _C, _H, _W = 2, 2, 224, 224
_INF = 1e12
_FALLBACK = (_H + _W) / 4.0


def _neighbor_sum(img):
    zr = jnp.zeros((_B, 1, _W), jnp.float32)
    zc = jnp.zeros((_B, _H, 1), jnp.float32)
    up = jnp.concatenate([img[:, 1:, :], zr], axis=1)
    down = jnp.concatenate([zr, img[:, :-1, :]], axis=1)
    left = jnp.concatenate([img[:, :, 1:], zc], axis=2)
    right = jnp.concatenate([zc, img[:, :, :-1]], axis=2)
    return img + up + down + left + right


def _hd_kernel(pred_ref, lab_ref, out_ref, pad_ref, run_ref):
    p0 = pred_ref[:, 0]
    p1 = pred_ref[:, 1]
    img_b = lab_ref[:, 0].astype(jnp.float32)
    img_a = (p1 > p0).astype(jnp.float32)

    boundary_a = img_a * (5.0 - _neighbor_sum(img_a)) * (1.0 - img_b)
    boundary_b = img_b * (5.0 - _neighbor_sum(img_b))
    ma = (boundary_a > 0.0).astype(jnp.float32)
    mb = (boundary_b > 0.0).astype(jnp.float32)

    inf_blk = jnp.full((_B, _H, _W), jnp.float32(_INF), jnp.float32)
    pad_ref[:, 0:_H, :] = inf_blk
    pad_ref[:, 2 * _H:3 * _H, :] = inf_blk

    d = jnp.where(mb > 0.0, jnp.float32(0.0), jnp.float32(_INF))
    for k in (1, 2, 4, 8, 16, 32, 64, 128):
        pad_ref[:, _H:2 * _H, :] = d
        up = pad_ref[:, _H - k:2 * _H - k, :]
        dn = pad_ref[:, _H + k:2 * _H + k, :]
        d = jnp.minimum(d, jnp.minimum(up, dn) + jnp.float32(k))
    d1 = d * d

    d1t = jnp.transpose(d1, (0, 2, 1))
    mat = jnp.transpose(ma, (0, 2, 1))
    pad_ref[:, _H:2 * _H, :] = d1t
    run_ref[...] = d1t

    mm0 = jnp.max(jnp.where(mat > 0.0, d1t, -jnp.float32(_INF)))

    def cond(carry):
        t, mm = carry
        s0 = 8 * t + 1
        return (s0 < _W) & (mm > (s0 * s0).astype(jnp.float32))

    def body(carry):
        t, _ = carry
        wp = pad_ref[:, pl.ds(8 * (_H // 8 - 1 - t), _H + 8), :]
        wm = pad_ref[:, pl.ds(_H + 8 * t, _H + 8), :]
        r = run_ref[...]
        for k in range(8):
            sf = (8 * t + (1 + k)).astype(jnp.float32)
            cand = jnp.minimum(wp[:, 7 - k:7 - k + _H, :],
                               wm[:, 1 + k:1 + k + _H, :])
            r = jnp.minimum(r, cand + sf * sf)
        run_ref[...] = r
        mm = jnp.max(jnp.where(mat > 0.0, r, -jnp.float32(_INF)))
        return t + 1, mm

    jax.lax.while_loop(cond, body, (jnp.int32(0), mm0))

    m = jnp.where(mat > 0.0, run_ref[...], -jnp.float32(_INF))
    hd2 = jnp.max(jnp.max(m, axis=2), axis=1, keepdims=True)
    any_a = jnp.max(jnp.max(ma, axis=2), axis=1, keepdims=True)
    any_b = jnp.max(jnp.max(mb, axis=2), axis=1, keepdims=True)

    hd = jnp.sqrt(jnp.maximum(hd2, 0.0))
    has_both = (any_a > 0.0) & (any_b > 0.0)
    val = jnp.where(has_both, hd, jnp.float32(_FALLBACK))

    lane = jax.lax.broadcasted_iota(jnp.int32, (_B, _C + 2), 1)
    out_ref[...] = jnp.where(lane == 1, val,
                             jnp.where(lane == 2, val * 0.5, jnp.float32(0.0)))


def kernel(predictions, labels):
    return pl.pallas_call(
        _hd_kernel,
        grid=(1,),
        in_specs=[
            pl.BlockSpec((_B, _C, _H, _W), lambda i: (0, 0, 0, 0)),
            pl.BlockSpec((_B, 1, _H, _W), lambda i: (0, 1, 0, 0)),
        ],
        out_specs=pl.BlockSpec((_B, _C + 2), lambda i: (0, 0)),
        out_shape=jax.ShapeDtypeStruct((_B, _C + 2), jnp.float32),
        scratch_shapes=[
            pltpu.VMEM((_B, 3 * _H, _W), jnp.float32),
            pltpu.VMEM((_B, _H, _W), jnp.float32),
        ],
    )(predictions, labels)


# speedup vs baseline: 1899.8018x; 1.3577x over previous
import jax
import jax.numpy as jnp
from jax.experimental import pallas as pl
from jax.experimental.pallas import tpu as pltpu

_B, _C, _H, _W = 2, 2, 224, 224
_INF = 1e12
_FALLBACK = (_H + _W) / 4.0


def _neighbor_sum(img):
    zr = jnp.zeros((1, _W), jnp.float32)
    zc = jnp.zeros((_H, 1), jnp.float32)
    up = jnp.concatenate([img[1:, :], zr], axis=0)
    down = jnp.concatenate([zr, img[:-1, :]], axis=0)
    left = jnp.concatenate([img[:, 1:], zc], axis=1)
    right = jnp.concatenate([zc, img[:, :-1]], axis=1)
    return img + up + down + left + right


def _hd_kernel(pred_ref, lab_ref, out_ref, pad_ref, run_ref):
    p0 = pred_ref[0, 0]
    p1 = pred_ref[0, 1]
    img_b = lab_ref[0, 0].astype(jnp.float32)
    img_a = (p1 > p0).astype(jnp.float32)

    boundary_a = img_a * (5.0 - _neighbor_sum(img_a)) * (1.0 - img_b)
    boundary_b = img_b * (5.0 - _neighbor_sum(img_b))
    ma = (boundary_a > 0.0).astype(jnp.float32)
    mb = (boundary_b > 0.0).astype(jnp.float32)

    inf_128 = jnp.full((128, _W), jnp.float32(_INF), jnp.float32)
    pad_ref[_H - 128:_H, :] = inf_128
    pad_ref[2 * _H:2 * _H + 128, :] = inf_128

    d = jnp.where(mb > 0.0, jnp.float32(0.0), jnp.float32(_INF))
    for k in (1, 2, 4, 8, 16, 32, 64, 128):
        pad_ref[_H:2 * _H, :] = d
        up = pad_ref[_H - k:2 * _H - k, :]
        dn = pad_ref[_H + k:2 * _H + k, :]
        d = jnp.minimum(d, jnp.minimum(up, dn) + jnp.float32(k))
    d1 = d * d

    d1t = d1.T
    mat = ma.T
    pad_ref[_H:2 * _H, :] = d1t

    w0 = pad_ref[_H - 4:_H + 228, :]
    r = d1t
    for s in (1, 2, 3, 4):
        cand = jnp.minimum(w0[4 - s:228 - s, :], w0[4 + s:228 + s, :])
        r = jnp.minimum(r, cand + jnp.float32(s * s))
    run_ref[...] = r
    mm1 = jnp.max(jnp.where(mat > 0.0, r, -jnp.float32(_INF)))

    @pl.when(mm1 > 25.0)
    def _():
        inf_96 = jnp.full((96, _W), jnp.float32(_INF), jnp.float32)
        pad_ref[0:96, :] = inf_96
        pad_ref[2 * _H + 128:3 * _H, :] = inf_96

    def cond(carry):
        t, mm = carry
        s0 = jnp.maximum(8 * t + 1, 5)
        return (8 * t + 1 < _W) & (mm > (s0 * s0).astype(jnp.float32))

    def body(carry):
        t, _ = carry
        wp = pad_ref[pl.ds(8 * (_H // 8 - 1 - t), _H + 8), :]
        wm = pad_ref[pl.ds(_H + 8 * t, _H + 8), :]
        r = run_ref[...]
        for k in range(8):
            sf = (8 * t + (1 + k)).astype(jnp.float32)
            cand = jnp.minimum(wp[7 - k:7 - k + _H, :],
                               wm[1 + k:1 + k + _H, :])
            r = jnp.minimum(r, cand + sf * sf)
        run_ref[...] = r
        mm = jnp.max(jnp.where(mat > 0.0, r, -jnp.float32(_INF)))
        return t + 1, mm

    _, mm_final = jax.lax.while_loop(cond, body, (jnp.int32(0), mm1))

    hd = jnp.sqrt(jnp.maximum(mm_final, 0.0))
    has_both = (mm_final >= 0.0) & (mm_final < 1e11)
    val = jnp.where(has_both, hd, jnp.float32(_FALLBACK))

    lane = jax.lax.broadcasted_iota(jnp.int32, (1, 1, _C + 2), 2)
    out_ref[...] = jnp.where(lane == 1, val,
                             jnp.where(lane == 2, val * 0.5, jnp.float32(0.0)))


def kernel(predictions, labels):
    res = pl.pallas_call(
        _hd_kernel,
        grid=(_B,),
        in_specs=[
            pl.BlockSpec((1, _C, _H, _W), lambda b: (b, 0, 0, 0)),
            pl.BlockSpec((1, 1, _H, _W), lambda b: (b, 1, 0, 0)),
        ],
        out_specs=pl.BlockSpec((1, 1, _C + 2), lambda b: (b, 0, 0)),
        out_shape=jax.ShapeDtypeStruct((_B, 1, _C + 2), jnp.float32),
        scratch_shapes=[
            pltpu.VMEM((3 * _H, _W), jnp.float32),
            pltpu.VMEM((_H, _W), jnp.float32),
        ],
    )(predictions, labels)
    return res.reshape(_B, _C + 2)
